# baseline (speedup 1.0000x reference)
import jax
import jax.numpy as jnp
from jax import lax
from jax.experimental import pallas as pl
from jax.experimental.pallas import tpu as pltpu
from jax.experimental.pallas import tpu_sc as plsc

NC = 2
NS = 16
NW = NC * NS

D = 33
CHUNK = 128
B_TOTAL = 16384 * 10
PER_W = B_TOTAL // NW
NCHUNK = PER_W // CHUNK
NBUF = 10
L1 = NBUF // 2
NROUND = NCHUNK // NBUF


def _body(idx_hbm, table_hbm, out_hbm, idx_v, rows_v, gsem, osem):
    wid = lax.axis_index("s") * NC + lax.axis_index("c")
    pltpu.sync_copy(idx_hbm.at[wid], idx_v)
    base = wid * PER_W

    def gather(j, s):
        pltpu.async_copy(table_hbm.at[idx_v.at[j]], rows_v.at[s], gsem.at[s])

    def out_copy(j, s):
        pltpu.async_copy(
            rows_v.at[s], out_hbm.at[pl.ds(base + j * CHUNK, CHUNK)], osem.at[s]
        )

    def wait_gather(s):
        pltpu.make_async_copy(
            table_hbm.at[idx_v.at[0]], rows_v.at[s], gsem.at[s]
        ).wait()

    def wait_out(s):
        pltpu.make_async_copy(
            rows_v.at[s], out_hbm.at[pl.ds(base, CHUNK)], osem.at[s]
        ).wait()

    for b in range(L1):
        gather(b, b)

    def round_fn(r, carry):
        for b in range(NBUF):
            j = r * NBUF + b
            s_new = (b + L1) % NBUF
            j_new = j + L1
            if b < L1:
                @pl.when(r >= 1)
                def _():
                    wait_out(s_new)
                    gather(j_new, s_new)

                @pl.when(r == 0)
                def _():
                    gather(j_new, s_new)
            else:
                @pl.when(r < NROUND - 1)
                def _():
                    wait_out(s_new)
                    gather(j_new, s_new)

            wait_gather(b)
            out_copy(j, b)
        return carry

    lax.fori_loop(0, NROUND, round_fn, 0)

    for b in range(NBUF):
        wait_out(b)


def kernel(idx, embedding):
    one = jnp.where(jnp.min(idx) > jnp.int32(-1), jnp.float32(1.0), jnp.float32(0.0))
    emb_lin = embedding * one
    idx3 = idx.reshape(NW, NCHUNK, CHUNK)
    mesh = plsc.VectorSubcoreMesh(
        core_axis_name="c", subcore_axis_name="s", num_cores=NC, num_subcores=NS
    )
    out = pl.kernel(
        _body,
        out_type=jax.ShapeDtypeStruct((B_TOTAL, D), jnp.float32),
        mesh=mesh,
        scratch_types=[
            pltpu.VMEM((NCHUNK, CHUNK), jnp.int32),
            pltpu.VMEM((NBUF, CHUNK, D), jnp.float32),
            pltpu.SemaphoreType.DMA((NBUF,)),
            pltpu.SemaphoreType.DMA((NBUF,)),
        ],
        compiler_params=pltpu.CompilerParams(use_tc_tiling_on_sc=False),
    )(idx3, emb_lin)
    return out.reshape(idx.shape[0], idx.shape[1], D)


# speedup vs baseline: 1.7454x; 1.7454x over previous
import jax
import jax.numpy as jnp
from jax import lax
from jax.experimental import pallas as pl
from jax.experimental.pallas import tpu as pltpu
from jax.experimental.pallas import tpu_sc as plsc

NC = 2
NS = 16
NW = NC * NS

D = 33
NROWS = 1000000
NGROUP = NROWS // 8
PAD_G = 64
PAD_STEPS = -(-NGROUP // PAD_G)
PAD_K = -(-PAD_STEPS // NW)

CHUNK = 128
B_TOTAL = 16384 * 10
PER_W = B_TOTAL // NW
NCHUNK = PER_W // CHUNK
NBUF = 5
L1 = 2
NROUND = NCHUNK // NBUF


def _pad_body(table_hbm, dep_hbm, buf_v, sem):
    wid = lax.axis_index("s") * NC + lax.axis_index("c")
    tbl3 = table_hbm.reshape(NGROUP, 8, D)

    def step(k, carry):
        s = k * NW + wid

        @pl.when(s < PAD_STEPS)
        def _():
            g0 = jnp.minimum(s * PAD_G, NGROUP - PAD_G)
            pltpu.async_copy(
                tbl3.at[pl.ds(g0, PAD_G)], buf_v.at[:, :, pl.ds(0, D)], sem
            ).wait()
            pltpu.sync_copy(
                buf_v.at[:, :, pl.ds(0, D)],
                dep_hbm.at[pl.ds(g0, PAD_G), :, pl.ds(0, D)],
            )

        return carry

    lax.fori_loop(0, PAD_K, step, 0)


def _gather_body(idx_hbm, dep_hbm, out_hbm, idx_v, rows_v, gsem, osem):
    wid = lax.axis_index("s") * NC + lax.axis_index("c")
    pltpu.sync_copy(idx_hbm.at[wid], idx_v)
    base = wid * PER_W
    tbl = dep_hbm

    def gather(j, s):
        pltpu.async_copy(tbl.at[idx_v.at[j]], rows_v.at[s], gsem.at[s])

    def out_copy(j, s):
        pltpu.async_copy(
            rows_v.at[s], out_hbm.at[pl.ds(base + j * CHUNK, CHUNK)], osem.at[s]
        )

    def wait_gather(s):
        pltpu.make_async_copy(tbl.at[idx_v.at[0]], rows_v.at[s], gsem.at[s]).wait()

    def wait_out(s):
        pltpu.make_async_copy(
            rows_v.at[s], out_hbm.at[pl.ds(base, CHUNK)], osem.at[s]
        ).wait()

    for b in range(L1):
        gather(b, b)

    def round_fn(r, carry):
        for b in range(NBUF):
            j = r * NBUF + b
            s_new = (b + L1) % NBUF
            j_new = j + L1
            if b < NBUF - L1:
                @pl.when(r >= 1)
                def _():
                    wait_out(s_new)
                    gather(j_new, s_new)

                @pl.when(r == 0)
                def _():
                    gather(j_new, s_new)
            else:
                @pl.when(r < NROUND - 1)
                def _():
                    wait_out(s_new)
                    gather(j_new, s_new)

            wait_gather(b)
            out_copy(j, b)
        return carry

    lax.fori_loop(0, NROUND, round_fn, 0)

    for b in range(NBUF):
        wait_out(b)


def kernel(idx, embedding):
    idx3 = idx.reshape(NW, NCHUNK, CHUNK)
    mesh = plsc.VectorSubcoreMesh(
        core_axis_name="c", subcore_axis_name="s", num_cores=NC, num_subcores=NS
    )
    dep = jnp.pad(embedding, ((0, 0), (0, 128 - D)))
    out128 = pl.kernel(
        _gather_body,
        out_type=jax.ShapeDtypeStruct((B_TOTAL, 128), jnp.float32),
        mesh=mesh,
        scratch_types=[
            pltpu.VMEM((NCHUNK, CHUNK), jnp.int32),
            pltpu.VMEM((NBUF, CHUNK, 128), jnp.float32),
            pltpu.SemaphoreType.DMA((NBUF,)),
            pltpu.SemaphoreType.DMA((NBUF,)),
        ],
    )(idx3, dep.reshape(NROWS, 128))
    return out128[:, :D].reshape(idx.shape[0], idx.shape[1], D)


# speedup vs baseline: 1.7463x; 1.0005x over previous
import jax
import jax.numpy as jnp
from jax import lax
from jax.experimental import pallas as pl
from jax.experimental.pallas import tpu as pltpu
from jax.experimental.pallas import tpu_sc as plsc

NC = 2
NS = 16
NW = NC * NS

D = 33
NROWS = 1000000
NGROUP = NROWS // 8
PAD_G = 64
PAD_STEPS = -(-NGROUP // PAD_G)
PAD_K = -(-PAD_STEPS // NW)

CHUNK = 128
B_TOTAL = 16384 * 10
PER_W = B_TOTAL // NW
NCHUNK = PER_W // CHUNK
NBUF = 5
L1 = 2
NROUND = NCHUNK // NBUF


def _pad_body(table_hbm, dep_hbm, buf_v, sem):
    wid = lax.axis_index("s") * NC + lax.axis_index("c")
    tbl3 = table_hbm.reshape(NGROUP, 8, D)

    def step(k, carry):
        s = k * NW + wid

        @pl.when(s < PAD_STEPS)
        def _():
            g0 = jnp.minimum(s * PAD_G, NGROUP - PAD_G)
            pltpu.async_copy(
                tbl3.at[pl.ds(g0, PAD_G)], buf_v.at[:, :, pl.ds(0, D)], sem
            ).wait()
            pltpu.sync_copy(
                buf_v.at[:, :, pl.ds(0, D)],
                dep_hbm.at[pl.ds(g0, PAD_G), :, pl.ds(0, D)],
            )

        return carry

    lax.fori_loop(0, PAD_K, step, 0)


def _gather_body(idx_hbm, dep_hbm, out_hbm, idx_v, rows_v, gsem, osem):
    wid = lax.axis_index("s") * NC + lax.axis_index("c")
    pltpu.sync_copy(idx_hbm.at[wid], idx_v)
    base = wid * PER_W
    tbl = dep_hbm

    def gather(j, s):
        pltpu.async_copy(tbl.at[idx_v.at[j]], rows_v.at[s], gsem.at[s])

    def out_copy(j, s):
        pltpu.async_copy(
            rows_v.at[s], out_hbm.at[pl.ds(base + j * CHUNK, CHUNK)], osem.at[s]
        )

    def wait_gather(s):
        pltpu.make_async_copy(tbl.at[idx_v.at[0]], rows_v.at[s], gsem.at[s]).wait()

    def wait_out(s):
        pltpu.make_async_copy(
            rows_v.at[s], out_hbm.at[pl.ds(base, CHUNK)], osem.at[s]
        ).wait()

    for b in range(L1):
        gather(b, b)

    def round_fn(r, carry):
        for b in range(NBUF):
            j = r * NBUF + b
            s_new = (b + L1) % NBUF
            j_new = j + L1
            if b < NBUF - L1:
                @pl.when(r >= 1)
                def _():
                    wait_out(s_new)
                    gather(j_new, s_new)

                @pl.when(r == 0)
                def _():
                    gather(j_new, s_new)
            else:
                @pl.when(r < NROUND - 1)
                def _():
                    wait_out(s_new)
                    gather(j_new, s_new)

            wait_gather(b)
            out_copy(j, b)
        return carry

    lax.fori_loop(0, NROUND, round_fn, 0)

    for b in range(NBUF):
        wait_out(b)


def kernel(idx, embedding):
    idx3 = idx.reshape(NW, NCHUNK, CHUNK)
    mesh = plsc.VectorSubcoreMesh(
        core_axis_name="c", subcore_axis_name="s", num_cores=NC, num_subcores=NS
    )
    dep = jnp.pad(embedding, ((0, 0), (0, 128 - D)))
    out128 = pl.kernel(
        _gather_body,
        out_type=jax.ShapeDtypeStruct((B_TOTAL, 128), jnp.float32),
        mesh=mesh,
        scratch_types=[
            pltpu.VMEM((NCHUNK, CHUNK), jnp.int32),
            pltpu.VMEM((NBUF, CHUNK, 128), jnp.float32),
            pltpu.SemaphoreType.DMA((NBUF,)),
            pltpu.SemaphoreType.DMA((NBUF,)),
        ],
        compiler_params=pltpu.CompilerParams(needs_layout_passes=True),
    )(idx3, dep.reshape(NROWS, 128))
    return out128[:, :D].reshape(idx.shape[0], idx.shape[1], D)
